# baseline (speedup 1.0000x reference)
import functools

import jax
import jax.numpy as jnp
from jax import lax
from jax.experimental import pallas as pl
from jax.experimental.pallas import tpu as pltpu
from jax.experimental.pallas import tpu_sc as plsc

N_FIELDS = 26
CARD = 10000
EMB_D = 101
D_MODEL = 128
BATCH = 16384
EPS = 1e-5

TOTAL = N_FIELDS * BATCH
BM = 2000

NC = 2
NS = 16
NW = NC * NS
PER_W = TOTAL // NW
B_PER_W = BATCH // NW
CH = 256
CH_PER_F = B_PER_W // CH
N_CHUNKS = N_FIELDS * CH_PER_F
LOG2_BATCH = 14


def _tc_project_body(tbl_ref, w_ref, b_ref, g_ref, be_ref, out_ref):
    emb = tbl_ref[0]
    w = w_ref[0]
    prj = jnp.dot(emb, w, preferred_element_type=jnp.float32)
    prj = prj + b_ref[0][0][None, :]
    mean = jnp.mean(prj, axis=-1, keepdims=True)
    cent = prj - mean
    var = jnp.mean(cent * cent, axis=-1, keepdims=True)
    inv = lax.rsqrt(var + EPS)
    out_ref[0] = cent * inv * g_ref[0][0][None, :] + be_ref[0][0][None, :]


EMB_P = 128


def _project_tables(tables, W, b, gamma, beta):
    tables_p = jnp.pad(tables, ((0, 0), (0, 0), (0, EMB_P - EMB_D)))
    W_p = jnp.pad(W, ((0, 0), (0, EMB_P - EMB_D), (0, 0)))
    b3 = b[:, None, :]
    g3 = gamma[:, None, :]
    be3 = beta[:, None, :]
    return pl.pallas_call(
        _tc_project_body,
        grid=(N_FIELDS, CARD // BM),
        in_specs=[
            pl.BlockSpec((1, BM, EMB_P), lambda f, m: (f, m, 0)),
            pl.BlockSpec((1, EMB_P, D_MODEL), lambda f, m: (f, 0, 0)),
            pl.BlockSpec((1, 1, D_MODEL), lambda f, m: (f, 0, 0)),
            pl.BlockSpec((1, 1, D_MODEL), lambda f, m: (f, 0, 0)),
            pl.BlockSpec((1, 1, D_MODEL), lambda f, m: (f, 0, 0)),
        ],
        out_specs=pl.BlockSpec((1, BM, D_MODEL), lambda f, m: (f, m, 0)),
        out_shape=jax.ShapeDtypeStruct((N_FIELDS, CARD, D_MODEL), jnp.float32),
    )(tables_p, W_p, b3, g3, be3)


BT = 2048


def _tc_transpose_body(x_ref, out_ref):
    xt = x_ref[...].T
    foff = lax.broadcasted_iota(jnp.int32, (N_FIELDS, BT), 0) * CARD
    out_ref[...] = xt + foff


def _global_indices(x):
    return pl.pallas_call(
        _tc_transpose_body,
        grid=(BATCH // BT,),
        in_specs=[pl.BlockSpec((BT, N_FIELDS), lambda m: (m, 0))],
        out_specs=pl.BlockSpec((N_FIELDS, BT), lambda m: (0, m)),
        out_shape=jax.ShapeDtypeStruct((N_FIELDS, BATCH), jnp.int32),
    )(x)


def _sc_gather_body(
    gidx_hbm, table_hbm, out_hbm, idx2_v, idx_v, rows_v, x_sem, g_sem, w_sem
):
    wid = lax.axis_index("s") * NC + lax.axis_index("c")
    b0 = wid * B_PER_W

    pltpu.sync_copy(gidx_hbm.at[:, pl.ds(b0, B_PER_W)], idx2_v)

    def flatten(k, carry):
        f = lax.shift_right_logical(k, 5)
        j = lax.bitwise_and(k, 31)
        idx_v[pl.ds(k * 16, 16)] = idx2_v[f, pl.ds(j * 16, 16)]
        return carry

    lax.fori_loop(0, N_FIELDS * (B_PER_W // 16), flatten, 0)

    def gather(j, buf):
        return pltpu.make_async_copy(
            table_hbm.at[idx_v.at[pl.ds(j * CH, CH)]], rows_v.at[buf], g_sem
        )

    def write(j, buf):
        f = lax.shift_right_logical(j, 1)
        h = lax.bitwise_and(j, 1)
        return pltpu.make_async_copy(
            rows_v.at[buf],
            out_hbm.at[pl.ds(f * BATCH + b0 + h * CH, CH)],
            w_sem,
        )

    gather(0, 0).start()
    gather(1, 1).start()
    gather(0, 0).wait()
    write(0, 0).start()
    gather(2, 2).start()

    def chunk(j, carry):
        b = lax.rem(j, 3)
        nb = lax.rem(j + 2, 3)
        gather(j, b).wait()
        write(j, b).start()
        write(j - 1, nb).wait()
        gather(j + 2, nb).start()
        return carry

    lax.fori_loop(1, N_CHUNKS - 2, chunk, 0)

    j = N_CHUNKS - 2
    b = j % 3
    gather(j, b).wait()
    write(j, b).start()
    write(j - 1, (j + 2) % 3).wait()
    j = N_CHUNKS - 1
    b = j % 3
    gather(j, b).wait()
    write(j, b).start()
    write(j - 1, (j + 2) % 3).wait()
    write(j, b).wait()


@functools.cache
def _make_sc_gather():
    return pl.kernel(
        _sc_gather_body,
        out_type=jax.ShapeDtypeStruct((TOTAL, D_MODEL), jnp.float32),
        mesh=plsc.VectorSubcoreMesh(core_axis_name="c", subcore_axis_name="s"),
        scratch_types=[
            pltpu.VMEM((N_FIELDS, B_PER_W), jnp.int32),
            pltpu.VMEM((N_FIELDS * B_PER_W,), jnp.int32),
            pltpu.VMEM((3, CH, D_MODEL), jnp.float32),
            pltpu.SemaphoreType.DMA,
            pltpu.SemaphoreType.DMA,
            pltpu.SemaphoreType.DMA,
        ],
    )


def kernel(x, tables, W, b, gamma, beta):
    norm_table = _project_tables(tables, W, b, gamma, beta)
    gidx = _global_indices(x)
    out = _make_sc_gather()(gidx, norm_table.reshape(N_FIELDS * CARD, D_MODEL))
    return out.reshape(N_FIELDS, BATCH, D_MODEL)


# speedup vs baseline: 2.0283x; 2.0283x over previous
import functools

import jax
import jax.numpy as jnp
from jax import lax
from jax.experimental import pallas as pl
from jax.experimental.pallas import tpu as pltpu
from jax.experimental.pallas import tpu_sc as plsc

N_FIELDS = 26
CARD = 10000
EMB_D = 101
D_MODEL = 128
BATCH = 16384
EPS = 1e-5

TOTAL = N_FIELDS * BATCH
BM = 2000

NC = 2
NS = 16
NW = NC * NS
PER_W = TOTAL // NW
B_PER_W = BATCH // NW
CH = 256
CH_PER_F = B_PER_W // CH
N_CHUNKS = N_FIELDS * CH_PER_F
LOG2_BATCH = 14


def _tc_project_body(tbl_ref, w_ref, b_ref, g_ref, be_ref, out_ref):
    emb = tbl_ref[0]
    w = w_ref[0]
    prj = jnp.dot(emb, w, preferred_element_type=jnp.float32)
    prj = prj + b_ref[0][0][None, :]
    mean = jnp.mean(prj, axis=-1, keepdims=True)
    cent = prj - mean
    var = jnp.mean(cent * cent, axis=-1, keepdims=True)
    inv = lax.rsqrt(var + EPS)
    out_ref[0] = cent * inv * g_ref[0][0][None, :] + be_ref[0][0][None, :]


def _project_tables(tables, W, b, gamma, beta):
    b3 = b[:, None, :]
    g3 = gamma[:, None, :]
    be3 = beta[:, None, :]
    return pl.pallas_call(
        _tc_project_body,
        grid=(N_FIELDS, CARD // BM),
        in_specs=[
            pl.BlockSpec((1, BM, EMB_D), lambda f, m: (f, m, 0)),
            pl.BlockSpec((1, EMB_D, D_MODEL), lambda f, m: (f, 0, 0)),
            pl.BlockSpec((1, 1, D_MODEL), lambda f, m: (f, 0, 0)),
            pl.BlockSpec((1, 1, D_MODEL), lambda f, m: (f, 0, 0)),
            pl.BlockSpec((1, 1, D_MODEL), lambda f, m: (f, 0, 0)),
        ],
        out_specs=pl.BlockSpec((1, BM, D_MODEL), lambda f, m: (f, m, 0)),
        out_shape=jax.ShapeDtypeStruct((N_FIELDS, CARD, D_MODEL), jnp.float32),
        compiler_params=pltpu.CompilerParams(
            allow_input_fusion=[True, False, False, False, False]
        ),
    )(tables, W, b3, g3, be3)


BT = 2048


def _tc_transpose_body(x_ref, out_ref):
    xt = x_ref[...].T
    foff = lax.broadcasted_iota(jnp.int32, (N_FIELDS, BT), 0) * CARD
    out_ref[...] = xt + foff


def _global_indices(x):
    return pl.pallas_call(
        _tc_transpose_body,
        grid=(BATCH // BT,),
        in_specs=[pl.BlockSpec((BT, N_FIELDS), lambda m: (m, 0))],
        out_specs=pl.BlockSpec((N_FIELDS, BT), lambda m: (0, m)),
        out_shape=jax.ShapeDtypeStruct((N_FIELDS, BATCH), jnp.int32),
    )(x)


def _sc_gather_body(
    gidx_hbm, table_hbm, out_hbm, idx2_v, idx_v, rows_v, x_sem, g_sem, w_sem
):
    wid = lax.axis_index("s") * NC + lax.axis_index("c")
    b0 = wid * B_PER_W

    pltpu.sync_copy(gidx_hbm.at[:, pl.ds(b0, B_PER_W)], idx2_v)

    def flatten(k, carry):
        f = lax.shift_right_logical(k, 5)
        j = lax.bitwise_and(k, 31)
        idx_v[pl.ds(k * 16, 16)] = idx2_v[f, pl.ds(j * 16, 16)]
        return carry

    lax.fori_loop(0, N_FIELDS * (B_PER_W // 16), flatten, 0)

    def gather(j, buf):
        return pltpu.make_async_copy(
            table_hbm.at[idx_v.at[pl.ds(j * CH, CH)]], rows_v.at[buf], g_sem
        )

    def write(j, buf):
        f = lax.shift_right_logical(j, 1)
        h = lax.bitwise_and(j, 1)
        return pltpu.make_async_copy(
            rows_v.at[buf],
            out_hbm.at[pl.ds(f * BATCH + b0 + h * CH, CH)],
            w_sem,
        )

    gather(0, 0).start()
    gather(1, 1).start()
    gather(0, 0).wait()
    write(0, 0).start()
    gather(2, 2).start()

    def chunk(j, carry):
        b = lax.rem(j, 3)
        nb = lax.rem(j + 2, 3)
        gather(j, b).wait()
        write(j, b).start()
        write(j - 1, nb).wait()
        gather(j + 2, nb).start()
        return carry

    lax.fori_loop(1, N_CHUNKS - 2, chunk, 0)

    j = N_CHUNKS - 2
    b = j % 3
    gather(j, b).wait()
    write(j, b).start()
    write(j - 1, (j + 2) % 3).wait()
    j = N_CHUNKS - 1
    b = j % 3
    gather(j, b).wait()
    write(j, b).start()
    write(j - 1, (j + 2) % 3).wait()
    write(j, b).wait()


@functools.cache
def _make_sc_gather():
    return pl.kernel(
        _sc_gather_body,
        out_type=jax.ShapeDtypeStruct((TOTAL, D_MODEL), jnp.float32),
        mesh=plsc.VectorSubcoreMesh(core_axis_name="c", subcore_axis_name="s"),
        scratch_types=[
            pltpu.VMEM((N_FIELDS, B_PER_W), jnp.int32),
            pltpu.VMEM((N_FIELDS * B_PER_W,), jnp.int32),
            pltpu.VMEM((3, CH, D_MODEL), jnp.float32),
            pltpu.SemaphoreType.DMA,
            pltpu.SemaphoreType.DMA,
            pltpu.SemaphoreType.DMA,
        ],
    )


def kernel(x, tables, W, b, gamma, beta):
    norm_table = _project_tables(tables, W, b, gamma, beta)
    gidx = _global_indices(x)
    out = _make_sc_gather()(gidx, norm_table.reshape(N_FIELDS * CARD, D_MODEL))
    return out.reshape(N_FIELDS, BATCH, D_MODEL)


# speedup vs baseline: 2.2721x; 1.1202x over previous
import functools

import jax
import jax.numpy as jnp
from jax import lax
from jax.experimental import pallas as pl
from jax.experimental.pallas import tpu as pltpu
from jax.experimental.pallas import tpu_sc as plsc

N_FIELDS = 26
CARD = 10000
EMB_D = 101
D_MODEL = 128
BATCH = 16384
EPS = 1e-5

TOTAL = N_FIELDS * BATCH
BM = 5000

NC = 2
NS = 16
NW = NC * NS
PER_W = TOTAL // NW
B_PER_W = BATCH // NW
CH = 256
CH_PER_F = B_PER_W // CH
N_CHUNKS = N_FIELDS * CH_PER_F
LOG2_BATCH = 14


def _tc_project_body(tbl_ref, w_ref, b_ref, g_ref, be_ref, out_ref):
    emb = tbl_ref[0]
    w = w_ref[0]
    prj = jnp.dot(emb, w, preferred_element_type=jnp.float32)
    prj = prj + b_ref[0][0][None, :]
    mean = jnp.mean(prj, axis=-1, keepdims=True)
    cent = prj - mean
    var = jnp.mean(cent * cent, axis=-1, keepdims=True)
    inv = lax.rsqrt(var + EPS)
    out_ref[0] = cent * inv * g_ref[0][0][None, :] + be_ref[0][0][None, :]


def _project_tables(tables, W, b, gamma, beta):
    b3 = b[:, None, :]
    g3 = gamma[:, None, :]
    be3 = beta[:, None, :]
    return pl.pallas_call(
        _tc_project_body,
        grid=(N_FIELDS, CARD // BM),
        in_specs=[
            pl.BlockSpec((1, BM, EMB_D), lambda f, m: (f, m, 0)),
            pl.BlockSpec((1, EMB_D, D_MODEL), lambda f, m: (f, 0, 0)),
            pl.BlockSpec((1, 1, D_MODEL), lambda f, m: (f, 0, 0)),
            pl.BlockSpec((1, 1, D_MODEL), lambda f, m: (f, 0, 0)),
            pl.BlockSpec((1, 1, D_MODEL), lambda f, m: (f, 0, 0)),
        ],
        out_specs=pl.BlockSpec((1, BM, D_MODEL), lambda f, m: (f, m, 0)),
        out_shape=jax.ShapeDtypeStruct((N_FIELDS, CARD, D_MODEL), jnp.float32),
        compiler_params=pltpu.CompilerParams(
            allow_input_fusion=[True, False, False, False, False]
        ),
    )(tables, W, b3, g3, be3)


BT = 2048


def _tc_transpose_body(x_ref, out_ref):
    xt = x_ref[...].T
    foff = lax.broadcasted_iota(jnp.int32, (N_FIELDS, BT), 0) * CARD
    out_ref[...] = xt + foff


def _global_indices(x):
    return pl.pallas_call(
        _tc_transpose_body,
        grid=(BATCH // BT,),
        in_specs=[pl.BlockSpec((BT, N_FIELDS), lambda m: (m, 0))],
        out_specs=pl.BlockSpec((N_FIELDS, BT), lambda m: (0, m)),
        out_shape=jax.ShapeDtypeStruct((N_FIELDS, BATCH), jnp.int32),
    )(x)


def _sc_gather_body(
    gidx_hbm, table_hbm, out_hbm, idx2_v, idx_v, rows_v, x_sem, g_sem, w_sem
):
    wid = lax.axis_index("s") * NC + lax.axis_index("c")
    b0 = wid * B_PER_W

    pltpu.sync_copy(gidx_hbm.at[:, pl.ds(b0, B_PER_W)], idx2_v)

    def flatten(k, carry):
        f = lax.shift_right_logical(k, 5)
        j = lax.bitwise_and(k, 31)
        idx_v[pl.ds(k * 16, 16)] = idx2_v[f, pl.ds(j * 16, 16)]
        return carry

    lax.fori_loop(0, N_FIELDS * (B_PER_W // 16), flatten, 0)

    def gather(j, buf):
        return pltpu.make_async_copy(
            table_hbm.at[idx_v.at[pl.ds(j * CH, CH)]], rows_v.at[buf], g_sem
        )

    def write(j, buf):
        f = lax.shift_right_logical(j, 1)
        h = lax.bitwise_and(j, 1)
        return pltpu.make_async_copy(
            rows_v.at[buf],
            out_hbm.at[pl.ds(f * BATCH + b0 + h * CH, CH)],
            w_sem,
        )

    gather(0, 0).start()
    gather(1, 1).start()
    gather(0, 0).wait()
    write(0, 0).start()
    gather(2, 2).start()

    def chunk(j, carry):
        b = lax.rem(j, 3)
        nb = lax.rem(j + 2, 3)
        gather(j, b).wait()
        write(j, b).start()
        write(j - 1, nb).wait()
        gather(j + 2, nb).start()
        return carry

    lax.fori_loop(1, N_CHUNKS - 2, chunk, 0)

    j = N_CHUNKS - 2
    b = j % 3
    gather(j, b).wait()
    write(j, b).start()
    write(j - 1, (j + 2) % 3).wait()
    j = N_CHUNKS - 1
    b = j % 3
    gather(j, b).wait()
    write(j, b).start()
    write(j - 1, (j + 2) % 3).wait()
    write(j, b).wait()


@functools.cache
def _make_sc_gather():
    return pl.kernel(
        _sc_gather_body,
        out_type=jax.ShapeDtypeStruct((TOTAL, D_MODEL), jnp.float32),
        mesh=plsc.VectorSubcoreMesh(core_axis_name="c", subcore_axis_name="s"),
        scratch_types=[
            pltpu.VMEM((N_FIELDS, B_PER_W), jnp.int32),
            pltpu.VMEM((N_FIELDS * B_PER_W,), jnp.int32),
            pltpu.VMEM((3, CH, D_MODEL), jnp.float32),
            pltpu.SemaphoreType.DMA,
            pltpu.SemaphoreType.DMA,
            pltpu.SemaphoreType.DMA,
        ],
    )


def kernel(x, tables, W, b, gamma, beta):
    norm_table = _project_tables(tables, W, b, gamma, beta)
    gidx = _global_indices(x)
    out = _make_sc_gather()(gidx, norm_table.reshape(N_FIELDS * CARD, D_MODEL))
    return out.reshape(N_FIELDS, BATCH, D_MODEL)


# speedup vs baseline: 2.3657x; 1.0412x over previous
import functools

import jax
import jax.numpy as jnp
from jax import lax
from jax.experimental import pallas as pl
from jax.experimental.pallas import tpu as pltpu
from jax.experimental.pallas import tpu_sc as plsc

N_FIELDS = 26
CARD = 10000
EMB_D = 101
D_MODEL = 128
BATCH = 16384
EPS = 1e-5

TOTAL = N_FIELDS * BATCH
BM = 10000

NC = 2
NS = 16
NW = NC * NS
PER_W = TOTAL // NW
B_PER_W = BATCH // NW
CH = 256
CH_PER_F = B_PER_W // CH
N_CHUNKS = N_FIELDS * CH_PER_F
LOG2_BATCH = 14


def _tc_project_body(tbl_ref, w_ref, b_ref, g_ref, be_ref, out_ref):
    emb = tbl_ref[0]
    w = w_ref[0]
    prj = jnp.dot(emb, w, preferred_element_type=jnp.float32)
    prj = prj + b_ref[0][0][None, :]
    mean = jnp.mean(prj, axis=-1, keepdims=True)
    cent = prj - mean
    var = jnp.mean(cent * cent, axis=-1, keepdims=True)
    inv = lax.rsqrt(var + EPS)
    out_ref[0] = cent * inv * g_ref[0][0][None, :] + be_ref[0][0][None, :]


def _project_tables(tables, W, b, gamma, beta):
    b3 = b[:, None, :]
    g3 = gamma[:, None, :]
    be3 = beta[:, None, :]
    return pl.pallas_call(
        _tc_project_body,
        grid=(N_FIELDS, CARD // BM),
        in_specs=[
            pl.BlockSpec((1, BM, EMB_D), lambda f, m: (f, m, 0)),
            pl.BlockSpec((1, EMB_D, D_MODEL), lambda f, m: (f, 0, 0)),
            pl.BlockSpec((1, 1, D_MODEL), lambda f, m: (f, 0, 0)),
            pl.BlockSpec((1, 1, D_MODEL), lambda f, m: (f, 0, 0)),
            pl.BlockSpec((1, 1, D_MODEL), lambda f, m: (f, 0, 0)),
        ],
        out_specs=pl.BlockSpec((1, BM, D_MODEL), lambda f, m: (f, m, 0)),
        out_shape=jax.ShapeDtypeStruct((N_FIELDS, CARD, D_MODEL), jnp.float32),
        compiler_params=pltpu.CompilerParams(
            allow_input_fusion=[True, False, False, False, False]
        ),
    )(tables, W, b3, g3, be3)


BT = 2048


def _tc_transpose_body(x_ref, out_ref):
    xt = x_ref[...].T
    foff = lax.broadcasted_iota(jnp.int32, (N_FIELDS, BT), 0) * CARD
    out_ref[...] = xt + foff


def _global_indices(x):
    return pl.pallas_call(
        _tc_transpose_body,
        grid=(BATCH // BT,),
        in_specs=[pl.BlockSpec((BT, N_FIELDS), lambda m: (m, 0))],
        out_specs=pl.BlockSpec((N_FIELDS, BT), lambda m: (0, m)),
        out_shape=jax.ShapeDtypeStruct((N_FIELDS, BATCH), jnp.int32),
    )(x)


def _sc_gather_body(
    gidx_hbm, table_hbm, out_hbm, idx2_v, idx_v, rows_v, x_sem, g_sem, w_sem
):
    wid = lax.axis_index("s") * NC + lax.axis_index("c")
    b0 = wid * B_PER_W

    pltpu.sync_copy(gidx_hbm.at[:, pl.ds(b0, B_PER_W)], idx2_v)

    def flatten(k, carry):
        f = lax.shift_right_logical(k, 5)
        j = lax.bitwise_and(k, 31)
        idx_v[pl.ds(k * 16, 16)] = idx2_v[f, pl.ds(j * 16, 16)]
        return carry

    lax.fori_loop(0, N_FIELDS * (B_PER_W // 16), flatten, 0)

    def gather(j, buf):
        return pltpu.make_async_copy(
            table_hbm.at[idx_v.at[pl.ds(j * CH, CH)]], rows_v.at[buf], g_sem
        )

    def write(j, buf):
        f = lax.shift_right_logical(j, 1)
        h = lax.bitwise_and(j, 1)
        return pltpu.make_async_copy(
            rows_v.at[buf],
            out_hbm.at[pl.ds(f * BATCH + b0 + h * CH, CH)],
            w_sem,
        )

    gather(0, 0).start()
    gather(1, 1).start()
    gather(0, 0).wait()
    write(0, 0).start()
    gather(2, 2).start()

    def chunk(j, carry):
        b = lax.rem(j, 3)
        nb = lax.rem(j + 2, 3)
        gather(j, b).wait()
        write(j, b).start()
        write(j - 1, nb).wait()
        gather(j + 2, nb).start()
        return carry

    lax.fori_loop(1, N_CHUNKS - 2, chunk, 0)

    j = N_CHUNKS - 2
    b = j % 3
    gather(j, b).wait()
    write(j, b).start()
    write(j - 1, (j + 2) % 3).wait()
    j = N_CHUNKS - 1
    b = j % 3
    gather(j, b).wait()
    write(j, b).start()
    write(j - 1, (j + 2) % 3).wait()
    write(j, b).wait()


@functools.cache
def _make_sc_gather():
    return pl.kernel(
        _sc_gather_body,
        out_type=jax.ShapeDtypeStruct((TOTAL, D_MODEL), jnp.float32),
        mesh=plsc.VectorSubcoreMesh(core_axis_name="c", subcore_axis_name="s"),
        scratch_types=[
            pltpu.VMEM((N_FIELDS, B_PER_W), jnp.int32),
            pltpu.VMEM((N_FIELDS * B_PER_W,), jnp.int32),
            pltpu.VMEM((3, CH, D_MODEL), jnp.float32),
            pltpu.SemaphoreType.DMA,
            pltpu.SemaphoreType.DMA,
            pltpu.SemaphoreType.DMA,
        ],
    )


def kernel(x, tables, W, b, gamma, beta):
    norm_table = _project_tables(tables, W, b, gamma, beta)
    gidx = _global_indices(x)
    out = _make_sc_gather()(gidx, norm_table.reshape(N_FIELDS * CARD, D_MODEL))
    return out.reshape(N_FIELDS, BATCH, D_MODEL)
